# baseline (speedup 1.0000x reference)
import functools

import jax
import jax.numpy as jnp
from jax.experimental import pallas as pl
from jax.experimental.pallas import tpu as pltpu

D = 128
BA = 128
BC = 128
_PAD = 256
_CELLS = 32
_SIDE = 188.0
_EPS = 1e-5
_MARGIN = 2.0


def _interleave_bits(v):
    v = (v | (v << 8)) & 0x00FF00FF
    v = (v | (v << 4)) & 0x0F0F0F0F
    v = (v | (v << 2)) & 0x33333333
    v = (v | (v << 1)) & 0x55555555
    return v


def _morton(ctrs):
    cs = _SIDE / _CELLS
    cx = jnp.clip((ctrs[:, 0] / cs).astype(jnp.int32), 0, _CELLS - 1)
    cy = jnp.clip((ctrs[:, 1] / cs).astype(jnp.int32), 0, _CELLS - 1)
    return _interleave_bits(cx) | (_interleave_bits(cy) << 1)


def _gnorm(x, g, b):
    mu = jnp.mean(x, axis=-1, keepdims=True)
    xc = x - mu
    var = jnp.mean(xc * xc, axis=-1, keepdims=True)
    return xc * jax.lax.rsqrt(var + _EPS) * g + b


def _cc_body(ctx_ref, w_ref, out_ref):
    out_ref[...] = jnp.dot(ctx_ref[...], w_ref[...],
                           preferred_element_type=jnp.float32)


def _fused_body(nctx, counts, jlist, th, agts, actr, cctr, cc,
                wq, gq, bq, wc1b, w1, b1, wd2, gd2, bd2, wc1a, gc1, bc1,
                wa, wc2, gng, gnb, wl, gl, bl, out):
    i = pl.program_id(0)
    a = agts[...]
    q = _gnorm(jnp.dot(a, wq[...], preferred_element_type=jnp.float32),
               gq[...], bq[...])
    q = jnp.maximum(q, 0.0)
    qc3 = jnp.dot(q, wc1b[...],
                  preferred_element_type=jnp.float32).reshape(BA, 1, D)
    ac = actr[...]
    axl = jnp.broadcast_to(ac[:, 0:1], (BA, D)).reshape(BA, 1, D)
    ayl = jnp.broadcast_to(ac[:, 1:2], (BA, D)).reshape(BA, 1, D)
    na23 = axl * axl + ayl * ayl
    axb = axl.astype(jnp.bfloat16).astype(jnp.float32)
    ayb = ayl.astype(jnp.bfloat16).astype(jnp.float32)
    thv = th[0, 0]
    th2 = thv * thv
    w1x = w1[0:1, :].reshape(1, 1, D)
    w1y = w1[1:2, :].reshape(1, 1, D)
    b13 = b1[...].reshape(1, 1, D)
    gc13 = gc1[...].reshape(1, 1, D)
    bc13 = bc1[...].reshape(1, 1, D)
    jofs3 = jax.lax.broadcasted_iota(jnp.int32, (1, BC, D), 1)

    def step(k, acc):
        j = jlist[i, k]
        off = j * BC
        cb = cctr[pl.ds(off, BC), :]
        cxl = jnp.broadcast_to(cb[:, 0:1], (BC, D)).reshape(1, BC, D)
        cyl = jnp.broadcast_to(cb[:, 1:2], (BC, D)).reshape(1, BC, D)
        nc23 = cxl * cxl + cyl * cyl
        cxb = cxl.astype(jnp.bfloat16).astype(jnp.float32)
        cyb = cyl.astype(jnp.bfloat16).astype(jnp.float32)
        d23 = (na23 + nc23) - 2.0 * (axb * cxb + ayb * cyb)
        m3 = jnp.logical_and(d23 <= th2, (jofs3 + off) < nctx)
        dx3 = axl - cxl
        dy3 = ayl - cyl
        h = jnp.maximum(dx3 * w1x + dy3 * w1y + b13, 0.0)
        t = jnp.dot(h.reshape(BA * BC, D).astype(jnp.bfloat16), wd2[...],
                    preferred_element_type=jnp.float32)
        dist = jnp.maximum(_gnorm(t, gd2[...], bd2[...]), 0.0)
        de = jnp.dot(dist.astype(jnp.bfloat16), wc1a[...],
                     preferred_element_type=jnp.float32)
        ccj = cc[pl.ds(off, BC), :].reshape(1, BC, D)
        u = de.reshape(BA, BC, D) + qc3 + ccj
        c = jnp.maximum(_gnorm(u, gc13, bc13), 0.0)
        c = jnp.where(m3, c, 0.0)
        return acc + jnp.sum(c, axis=1)

    n = counts[0, i]
    s = jax.lax.fori_loop(0, n, step, jnp.zeros((BA, D), jnp.float32))
    o = (jnp.dot(a, wa[...], preferred_element_type=jnp.float32)
         + jnp.dot(s, wc2[...], preferred_element_type=jnp.float32))
    o = jnp.maximum(_gnorm(o, gng[...], gnb[...]), 0.0)
    o = _gnorm(jnp.dot(o, wl[...], preferred_element_type=jnp.float32),
               gl[...], bl[...]) + a
    out[...] = jnp.maximum(o, 0.0)


def _pad_perm(keys, npad):
    perm = jnp.argsort(keys)
    n = keys.shape[0]
    return jnp.concatenate(
        [perm, jnp.full((npad - n,), perm[-1], perm.dtype)]), perm


def kernel(agts, agt_ctrs, ctx, ctx_ctrs, Wd1, bd1, Wd2, gd2, bd2, Wq, gq,
           bq, Wc1, gc1, bc1, Wc2, Wa, gn_g, gn_b, Wl, gl, bl, dist_th):
    na = agts.shape[0]
    nctx = ctx.shape[0]
    npa = -(-na // _PAD) * _PAD
    npc = -(-nctx // _PAD) * _PAD
    nbi = npa // BA
    nbj = npc // BC

    perm_a, perm_a0 = _pad_perm(_morton(agt_ctrs), npa)
    perm_c, _ = _pad_perm(_morton(ctx_ctrs), npc)
    agts_s = jnp.take(agts, perm_a, axis=0)
    actr_s = jnp.take(agt_ctrs, perm_a, axis=0)
    ctx_s = jnp.take(ctx, perm_c, axis=0)
    cctr_s = jnp.take(ctx_ctrs, perm_c, axis=0)
    inv_perm = jnp.zeros((na,), jnp.int32).at[perm_a0].set(
        jnp.arange(na, dtype=jnp.int32))

    ab = actr_s.reshape(nbi, BA, 2)
    cb = cctr_s.reshape(nbj, BC, 2)
    amin, amax = ab.min(axis=1), ab.max(axis=1)
    cmin, cmax = cb.min(axis=1), cb.max(axis=1)
    gap = jnp.maximum(
        0.0, jnp.maximum(amin[:, None, :] - cmax[None, :, :],
                         cmin[None, :, :] - amax[:, None, :]))
    absa = jnp.abs(ab).max(axis=1)
    absc = jnp.abs(cb).max(axis=1)
    marg = (absa[:, None, 0] * absc[None, :, 0]
            + absa[:, None, 1] * absc[None, :, 1]) * (2.0 ** -7) + _MARGIN
    act = jnp.sum(gap * gap, axis=-1) <= dist_th * dist_th + marg
    counts = jnp.sum(act, axis=1, dtype=jnp.int32).reshape(1, nbi)
    jlist = jnp.argsort(jnp.logical_not(act), axis=1,
                        stable=True).astype(jnp.int32)

    cc = pl.pallas_call(
        _cc_body,
        grid=(nbj,),
        in_specs=[pl.BlockSpec((BC, D), lambda i: (i, 0)),
                  pl.BlockSpec((D, D), lambda i: (0, 0))],
        out_specs=pl.BlockSpec((BC, D), lambda i: (i, 0)),
        out_shape=jax.ShapeDtypeStruct((npc, D), jnp.float32),
    )(ctx_s, Wc1[:, 2 * D:].T)

    th = jnp.reshape(dist_th, (1, 1)).astype(jnp.float32)
    smem = pl.BlockSpec(memory_space=pltpu.SMEM)
    whole = pl.BlockSpec((D, D), lambda i: (0, 0))
    vec = pl.BlockSpec((1, D), lambda i: (0, 0))

    out_s = pl.pallas_call(
        functools.partial(_fused_body, nctx),
        grid=(nbi,),
        in_specs=[
            smem,
            smem,
            smem,
            pl.BlockSpec((BA, D), lambda i: (i, 0)),
            pl.BlockSpec((BA, 2), lambda i: (i, 0)),
            pl.BlockSpec((npc, 2), lambda i: (0, 0)),
            pl.BlockSpec((npc, D), lambda i: (0, 0)),
            whole, vec, vec,
            whole,
            pl.BlockSpec((2, D), lambda i: (0, 0)),
            vec,
            whole, vec, vec,
            whole, vec, vec,
            whole, whole,
            vec, vec,
            whole, vec, vec,
        ],
        out_specs=pl.BlockSpec((BA, D), lambda i: (i, 0)),
        out_shape=jax.ShapeDtypeStruct((npa, D), jnp.float32),
        compiler_params=pltpu.CompilerParams(
            dimension_semantics=("arbitrary",),
            vmem_limit_bytes=100 * 1024 * 1024,
        ),
    )(counts, jlist, th, agts_s, actr_s, cctr_s, cc,
      Wq.T, gq.reshape(1, D), bq.reshape(1, D),
      Wc1[:, D:2 * D].T,
      Wd1.T, bd1.reshape(1, D),
      Wd2.T.astype(jnp.bfloat16), gd2.reshape(1, D), bd2.reshape(1, D),
      Wc1[:, :D].T.astype(jnp.bfloat16), gc1.reshape(1, D), bc1.reshape(1, D),
      Wa.T, Wc2.T,
      gn_g.reshape(1, D), gn_b.reshape(1, D),
      Wl.T, gl.reshape(1, D), bl.reshape(1, D))

    return jnp.take(out_s, inv_perm, axis=0)


# speedup vs baseline: 1.0241x; 1.0241x over previous
import functools

import jax
import jax.numpy as jnp
from jax import lax
from jax.experimental import pallas as pl
from jax.experimental.pallas import tpu as pltpu
from jax.experimental.pallas import tpu_sc as plsc

D = 128
BA = 128
BC = 128
_PAD = 256
_CELLS = 32
_SIDE = 188.0
_EPS = 1e-5
_MARGIN = 2.0


def _interleave_bits(v):
    v = (v | (v << 8)) & 0x00FF00FF
    v = (v | (v << 4)) & 0x0F0F0F0F
    v = (v | (v << 2)) & 0x33333333
    v = (v | (v << 1)) & 0x55555555
    return v


def _morton(ctrs):
    cs = _SIDE / _CELLS
    cx = jnp.clip((ctrs[:, 0] / cs).astype(jnp.int32), 0, _CELLS - 1)
    cy = jnp.clip((ctrs[:, 1] / cs).astype(jnp.int32), 0, _CELLS - 1)
    return _interleave_bits(cx) | (_interleave_bits(cy) << 1)


def _gnorm(x, g, b):
    mu = jnp.mean(x, axis=-1, keepdims=True)
    xc = x - mu
    var = jnp.mean(xc * xc, axis=-1, keepdims=True)
    return xc * jax.lax.rsqrt(var + _EPS) * g + b


def _cc_body(ctx_ref, w_ref, out_ref):
    out_ref[...] = jnp.dot(ctx_ref[...], w_ref[...],
                           preferred_element_type=jnp.float32)


def _fused_body(nctx, counts, jlist, th, agts, actr, cctr, cc,
                wq, gq, bq, wc1b, w1, b1, wd2, gd2, bd2, wc1a, gc1, bc1,
                wa, wc2, gng, gnb, wl, gl, bl, out):
    i = pl.program_id(0)
    a = agts[...]
    q = _gnorm(jnp.dot(a, wq[...], preferred_element_type=jnp.float32),
               gq[...], bq[...])
    q = jnp.maximum(q, 0.0)
    qc3 = jnp.dot(q, wc1b[...],
                  preferred_element_type=jnp.float32).reshape(BA, 1, D)
    ac = actr[...]
    axl = jnp.broadcast_to(ac[:, 0:1], (BA, D)).reshape(BA, 1, D)
    ayl = jnp.broadcast_to(ac[:, 1:2], (BA, D)).reshape(BA, 1, D)
    na23 = axl * axl + ayl * ayl
    axb = axl.astype(jnp.bfloat16).astype(jnp.float32)
    ayb = ayl.astype(jnp.bfloat16).astype(jnp.float32)
    thv = th[0, 0]
    th2 = thv * thv
    w1x = w1[0:1, :].reshape(1, 1, D)
    w1y = w1[1:2, :].reshape(1, 1, D)
    b13 = b1[...].reshape(1, 1, D)
    gc13 = gc1[...].reshape(1, 1, D)
    bc13 = bc1[...].reshape(1, 1, D)
    jofs3 = jax.lax.broadcasted_iota(jnp.int32, (1, BC, D), 1)

    def step(k, acc):
        j = jlist[i, k]
        off = j * BC
        cb = cctr[pl.ds(off, BC), :]
        cxl = jnp.broadcast_to(cb[:, 0:1], (BC, D)).reshape(1, BC, D)
        cyl = jnp.broadcast_to(cb[:, 1:2], (BC, D)).reshape(1, BC, D)
        nc23 = cxl * cxl + cyl * cyl
        cxb = cxl.astype(jnp.bfloat16).astype(jnp.float32)
        cyb = cyl.astype(jnp.bfloat16).astype(jnp.float32)
        d23 = (na23 + nc23) - 2.0 * (axb * cxb + ayb * cyb)
        m3 = jnp.logical_and(d23 <= th2, (jofs3 + off) < nctx)
        dx3 = axl - cxl
        dy3 = ayl - cyl
        h = jnp.maximum(dx3 * w1x + dy3 * w1y + b13, 0.0)
        t = jnp.dot(h.reshape(BA * BC, D), wd2[...],
                    preferred_element_type=jnp.float32)
        dist = jnp.maximum(_gnorm(t, gd2[...], bd2[...]), 0.0)
        de = jnp.dot(dist, wc1a[...], preferred_element_type=jnp.float32)
        ccj = cc[pl.ds(off, BC), :].reshape(1, BC, D)
        u = de.reshape(BA, BC, D) + qc3 + ccj
        c = jnp.maximum(_gnorm(u, gc13, bc13), 0.0)
        c = jnp.where(m3, c, 0.0)
        return acc + jnp.sum(c, axis=1)

    n = counts[0, i]
    s = jax.lax.fori_loop(0, n, step, jnp.zeros((BA, D), jnp.float32))
    o = (jnp.dot(a, wa[...], preferred_element_type=jnp.float32)
         + jnp.dot(s, wc2[...], preferred_element_type=jnp.float32))
    o = jnp.maximum(_gnorm(o, gng[...], gnb[...]), 0.0)
    o = _gnorm(jnp.dot(o, wl[...], preferred_element_type=jnp.float32),
               gl[...], bl[...]) + a
    out[...] = jnp.maximum(o, 0.0)


def _sc_gather_rows(table, idx):
    b = idx.shape[0]
    info = plsc.get_sparse_core_info()
    nw = info.num_cores * info.num_subcores
    bpw = b // nw
    mesh = plsc.VectorSubcoreMesh(core_axis_name="c", subcore_axis_name="s")

    @functools.partial(
        pl.kernel, mesh=mesh,
        out_type=jax.ShapeDtypeStruct((b, D), jnp.float32),
        scratch_types=[
            pltpu.VMEM((bpw,), jnp.int32),
            pltpu.VMEM((bpw, D), jnp.float32),
            pltpu.SemaphoreType.DMA,
        ],
    )
    def k(table_hbm, idx_hbm, out_hbm, idx_v, rows_v, sem):
        wid = lax.axis_index("s") * info.num_cores + lax.axis_index("c")
        base = wid * bpw
        pltpu.sync_copy(idx_hbm.at[pl.ds(base, bpw)], idx_v)
        pltpu.async_copy(table_hbm.at[idx_v], rows_v, sem).wait()
        pltpu.sync_copy(rows_v, out_hbm.at[pl.ds(base, bpw)])

    return k(table, idx)


def _pad_perm(keys, npad):
    perm = jnp.argsort(keys)
    n = keys.shape[0]
    return jnp.concatenate(
        [perm, jnp.full((npad - n,), perm[-1], perm.dtype)]), perm


def kernel(agts, agt_ctrs, ctx, ctx_ctrs, Wd1, bd1, Wd2, gd2, bd2, Wq, gq,
           bq, Wc1, gc1, bc1, Wc2, Wa, gn_g, gn_b, Wl, gl, bl, dist_th):
    na = agts.shape[0]
    nctx = ctx.shape[0]
    npa = -(-na // _PAD) * _PAD
    npc = -(-nctx // _PAD) * _PAD
    nbi = npa // BA
    nbj = npc // BC

    perm_a, perm_a0 = _pad_perm(_morton(agt_ctrs), npa)
    perm_c, _ = _pad_perm(_morton(ctx_ctrs), npc)
    agts_s = _sc_gather_rows(agts, perm_a)
    actr_s = jnp.take(agt_ctrs, perm_a, axis=0)
    ctx_s = _sc_gather_rows(ctx, perm_c)
    cctr_s = jnp.take(ctx_ctrs, perm_c, axis=0)
    inv_perm = jnp.zeros((na,), jnp.int32).at[perm_a0].set(
        jnp.arange(na, dtype=jnp.int32))

    ab = actr_s.reshape(nbi, BA, 2)
    cb = cctr_s.reshape(nbj, BC, 2)
    amin, amax = ab.min(axis=1), ab.max(axis=1)
    cmin, cmax = cb.min(axis=1), cb.max(axis=1)
    gap = jnp.maximum(
        0.0, jnp.maximum(amin[:, None, :] - cmax[None, :, :],
                         cmin[None, :, :] - amax[:, None, :]))
    absa = jnp.abs(ab).max(axis=1)
    absc = jnp.abs(cb).max(axis=1)
    marg = (absa[:, None, 0] * absc[None, :, 0]
            + absa[:, None, 1] * absc[None, :, 1]) * (2.0 ** -7) + _MARGIN
    act = jnp.sum(gap * gap, axis=-1) <= dist_th * dist_th + marg
    counts = jnp.sum(act, axis=1, dtype=jnp.int32).reshape(1, nbi)
    jlist = jnp.argsort(jnp.logical_not(act), axis=1,
                        stable=True).astype(jnp.int32)

    cc = pl.pallas_call(
        _cc_body,
        grid=(nbj,),
        in_specs=[pl.BlockSpec((BC, D), lambda i: (i, 0)),
                  pl.BlockSpec((D, D), lambda i: (0, 0))],
        out_specs=pl.BlockSpec((BC, D), lambda i: (i, 0)),
        out_shape=jax.ShapeDtypeStruct((npc, D), jnp.float32),
    )(ctx_s, Wc1[:, 2 * D:].T)

    th = jnp.reshape(dist_th, (1, 1)).astype(jnp.float32)
    smem = pl.BlockSpec(memory_space=pltpu.SMEM)
    whole = pl.BlockSpec((D, D), lambda i: (0, 0))
    vec = pl.BlockSpec((1, D), lambda i: (0, 0))

    out_s = pl.pallas_call(
        functools.partial(_fused_body, nctx),
        grid=(nbi,),
        in_specs=[
            smem,
            smem,
            smem,
            pl.BlockSpec((BA, D), lambda i: (i, 0)),
            pl.BlockSpec((BA, 2), lambda i: (i, 0)),
            pl.BlockSpec((npc, 2), lambda i: (0, 0)),
            pl.BlockSpec((npc, D), lambda i: (0, 0)),
            whole, vec, vec,
            whole,
            pl.BlockSpec((2, D), lambda i: (0, 0)),
            vec,
            whole, vec, vec,
            whole, vec, vec,
            whole, whole,
            vec, vec,
            whole, vec, vec,
        ],
        out_specs=pl.BlockSpec((BA, D), lambda i: (i, 0)),
        out_shape=jax.ShapeDtypeStruct((npa, D), jnp.float32),
        compiler_params=pltpu.CompilerParams(
            dimension_semantics=("arbitrary",),
            vmem_limit_bytes=100 * 1024 * 1024,
        ),
    )(counts, jlist, th, agts_s, actr_s, cctr_s, cc,
      Wq.T, gq.reshape(1, D), bq.reshape(1, D),
      Wc1[:, D:2 * D].T,
      Wd1.T, bd1.reshape(1, D),
      Wd2.T, gd2.reshape(1, D), bd2.reshape(1, D),
      Wc1[:, :D].T, gc1.reshape(1, D), bc1.reshape(1, D),
      Wa.T, Wc2.T,
      gn_g.reshape(1, D), gn_b.reshape(1, D),
      Wl.T, gl.reshape(1, D), bl.reshape(1, D))

    inv_pad = jnp.concatenate(
        [inv_perm, jnp.zeros((npa - na,), jnp.int32)])
    return _sc_gather_rows(out_s, inv_pad)[:na]
